# baseline (speedup 1.0000x reference)
import jax
import jax.numpy as jnp
from jax import lax
from jax.experimental import pallas as pl

N_NODES = 50000
K = 20
NPOLY = 2500
B = 25
T = 100
IN_C = 8
HID = 64
NPOLY_P = 2560
NPB = 512
GRID = NPOLY_P // NPB
NEG = -1e30
F32 = jnp.float32


def _mlp(X, W1a, b1a, g, bt, W2, b2, ones_ref):
    ha = jnp.dot(X, W1a, preferred_element_type=F32) + b1a
    d = ha[:, :HID] - ha[:, HID:]
    msq = jnp.dot(d * d, ones_ref, preferred_element_type=F32)
    h = jnp.maximum(d / jnp.sqrt(msq + 1e-5) * g + bt, 0.0)
    return jnp.dot(h, W2, preferred_element_type=F32) + b2


def _agg_concat(h, c):
    h3 = h.reshape(K, NPB, c)
    m1 = jnp.max(h3, axis=0, keepdims=True)
    ismax = h3 == m1
    cnt = jnp.sum(jnp.where(ismax, 1.0, 0.0), axis=0, keepdims=True)
    m2x = jnp.max(jnp.where(ismax, NEG, h3), axis=0, keepdims=True)
    m2 = jnp.where(cnt > 1.5, m1, m2x)
    agg = jnp.where(ismax, m2, m1)
    out = jnp.concatenate([h3, agg], axis=2)
    return out.reshape(K * NPB, 2 * c)


def _subgraph_kernel(xp_ref, ones_ref,
                     w10_ref, b10_ref, g0_ref, bt0_ref, w20_ref, b20_ref,
                     w11_ref, b11_ref, g1_ref, bt1_ref, w21_ref, b21_ref,
                     w12_ref, b12_ref, g2_ref, bt2_ref, w22_ref, b22_ref,
                     poly_ref, csq_ref):
    X = xp_ref[...].reshape(K * NPB, IN_C)
    ones = ones_ref[...]

    h = _mlp(X, w10_ref[...], b10_ref[...], g0_ref[...], bt0_ref[...],
             w20_ref[...], b20_ref[...], ones)
    X = _agg_concat(h, IN_C)

    h = _mlp(X, w11_ref[...], b11_ref[...], g1_ref[...], bt1_ref[...],
             w21_ref[...], b21_ref[...], ones)
    X = _agg_concat(h, 2 * IN_C)

    h = _mlp(X, w12_ref[...], b12_ref[...], g2_ref[...], bt2_ref[...],
             w22_ref[...], b22_ref[...], ones)
    m1 = jnp.max(h.reshape(K, NPB, 4 * IN_C), axis=0)

    pt = jnp.concatenate([m1, m1], axis=1)
    poly_ref[0] = pt

    cid = (lax.broadcasted_iota(jnp.int32, (NPB, 1), 0)
           + pl.program_id(0) * NPB)
    s = jnp.sum(jnp.where(cid < NPOLY, pt * pt, 0.0),
                axis=0, keepdims=True)

    @pl.when(pl.program_id(0) == 0)
    def _():
        csq_ref[...] = s

    @pl.when(pl.program_id(0) != 0)
    def _():
        csq_ref[...] += s


def _attn_kernel(p_ref, csq_ref, m_ref,
                 qw_ref, qb_ref, kw_ref, kb_ref, vw_ref, vb_ref, o_ref):
    p = p_ref[0]
    cn = jnp.sqrt(csq_ref[...])
    pn = p / cn
    rs = jnp.sum(pn * pn, axis=1, keepdims=True)
    pn = pn / jnp.maximum(jnp.sqrt(rs), 1e-12)

    Q = jnp.dot(pn, qw_ref[...], preferred_element_type=F32) + qb_ref[...]
    Kt = jnp.dot(pn, kw_ref[...], preferred_element_type=F32) + kb_ref[...]
    V = jnp.dot(pn, vw_ref[...], preferred_element_type=F32) + vb_ref[...]

    S = lax.dot_general(Q, Kt, (((1,), (1,)), ((), ())),
                        preferred_element_type=F32)
    S = jnp.where(m_ref[0] > 0.5, S, -1e9)
    mx = jnp.max(S, axis=1, keepdims=True)
    E = jnp.exp(S - mx)
    A = E / jnp.sum(E, axis=1, keepdims=True)
    o_ref[0] = jnp.dot(A, V, preferred_element_type=F32)


def kernel(x, edge_index, cluster, valid_len, time_step_len,
           sg0_W1, sg0_b1, sg0_g, sg0_bt, sg0_W2, sg0_b2,
           sg1_W1, sg1_b1, sg1_g, sg1_bt, sg1_W2, sg1_b2,
           sg2_W1, sg2_b1, sg2_g, sg2_bt, sg2_W2, sg2_b2,
           q_W, q_b, k_W, k_b, v_W, v_b):
    xp = jnp.pad(x.reshape(NPOLY, K, IN_C).transpose(1, 0, 2),
                 ((0, 0), (0, NPOLY_P - NPOLY), (0, 0)))

    r2 = lambda a: a.reshape(1, -1)
    wspec = lambda a: pl.BlockSpec(a.shape, lambda b: (0,) * a.ndim)
    ones_m = jnp.full((HID, HID), 1.0 / HID, F32)
    aw = lambda w: jnp.concatenate(
        [w, jnp.tile(jnp.mean(w, axis=1, keepdims=True), (1, HID))], axis=1)
    ab = lambda b: jnp.concatenate(
        [b, jnp.full((HID,), jnp.mean(b), F32)]).reshape(1, -1)
    weights1 = [aw(sg0_W1), ab(sg0_b1), r2(sg0_g), r2(sg0_bt), sg0_W2, r2(sg0_b2),
                aw(sg1_W1), ab(sg1_b1), r2(sg1_g), r2(sg1_bt), sg1_W2, r2(sg1_b2),
                aw(sg2_W1), ab(sg2_b1), r2(sg2_g), r2(sg2_bt), sg2_W2, r2(sg2_b2)]

    poly, csq = pl.pallas_call(
        _subgraph_kernel,
        grid=(GRID,),
        in_specs=[pl.BlockSpec((K, NPB, IN_C), lambda b: (0, b, 0)),
                  wspec(ones_m)]
                 + [wspec(w) for w in weights1],
        out_specs=[pl.BlockSpec((1, NPB, HID), lambda b: (b, 0, 0)),
                   pl.BlockSpec((1, HID), lambda b: (0, 0))],
        out_shape=[jax.ShapeDtypeStruct((GRID, NPB, HID), F32),
                   jax.ShapeDtypeStruct((1, HID), F32)],
    )(xp, ones_m, *weights1)

    vl = jnp.minimum(valid_len, time_step_len)
    keep = (jnp.arange(T, dtype=jnp.int32)[None, :] < vl[:, None])
    keep = keep.astype(F32).reshape(B, 1, T)

    weights2 = [q_W, r2(q_b), k_W, r2(k_b), v_W, r2(v_b)]
    out = pl.pallas_call(
        _attn_kernel,
        grid=(B,),
        in_specs=[pl.BlockSpec((1, T, HID), lambda b: (b, 0, 0)),
                  pl.BlockSpec((1, HID), lambda b: (0, 0)),
                  pl.BlockSpec((1, 1, T), lambda b: (b, 0, 0))]
                 + [wspec(w) for w in weights2],
        out_specs=pl.BlockSpec((1, T, HID), lambda b: (b, 0, 0)),
        out_shape=jax.ShapeDtypeStruct((B, T, HID), F32),
    )(poly.reshape(NPOLY_P, HID)[:NPOLY].reshape(B, T, HID),
      csq, keep, *weights2)

    return out


# speedup vs baseline: 1.1688x; 1.1688x over previous
import jax
import jax.numpy as jnp
from jax import lax
from jax.experimental import pallas as pl

N_NODES = 50000
K = 20
NPOLY = 2500
B = 25
T = 100
IN_C = 8
HID = 64
NPOLY_P = 2560
NPB = 512
GRID = NPOLY_P // NPB
NEG = -1e30
F32 = jnp.float32


def _mlp(X, W1a, g, bt, W2, b2, ones_ref):
    ha = jnp.dot(X, W1a, preferred_element_type=F32)
    d = ha[:, :HID] - ha[:, HID:]
    msq = jnp.dot(d * d, ones_ref, preferred_element_type=F32)
    v = msq + 1e-5
    r = lax.rsqrt(v)
    r = r * (1.5 - 0.5 * v * r * r)
    h = jnp.maximum(d * r * g + bt, 0.0)
    return jnp.dot(h, W2, preferred_element_type=F32) + b2


def _top2(h3):
    a, b = h3[:K // 2], h3[K // 2:]
    m1s = jnp.maximum(a, b)
    m2s = jnp.minimum(a, b)
    n = K // 2
    while n > 1:
        half = n // 2
        a1, b1 = m1s[:half], m1s[half:2 * half]
        a2, b2 = m2s[:half], m2s[half:2 * half]
        nm1 = jnp.maximum(a1, b1)
        nm2 = jnp.maximum(jnp.minimum(a1, b1), jnp.maximum(a2, b2))
        if n % 2:
            nm1 = jnp.concatenate([nm1, m1s[2 * half:]], axis=0)
            nm2 = jnp.concatenate([nm2, m2s[2 * half:]], axis=0)
        m1s, m2s = nm1, nm2
        n = half + (n % 2)
    return m1s, m2s


def _agg_concat(h, c):
    h3 = h.reshape(K, NPB, c)
    m1, m2 = _top2(h3)
    agg = jnp.where(h3 == m1, m2, m1)
    ones_ch = jnp.full((1, NPB, 1), 1.0, F32)
    out = jnp.concatenate(
        [h3, agg, jnp.broadcast_to(ones_ch, (K, NPB, 1))], axis=2)
    return out.reshape(K * NPB, 2 * c + 1)


def _subgraph_kernel(xp_ref, ones_ref,
                     w10_ref, g0_ref, bt0_ref, w20_ref, b20_ref,
                     w11_ref, g1_ref, bt1_ref, w21_ref, b21_ref,
                     w12_ref, g2_ref, bt2_ref, w22_ref, b22_ref,
                     poly_ref, csq_ref):
    X = xp_ref[...].reshape(K * NPB, IN_C + 1)
    ones = ones_ref[...]

    h = _mlp(X, w10_ref[...], g0_ref[...], bt0_ref[...],
             w20_ref[...], b20_ref[...], ones)
    X = _agg_concat(h, IN_C)

    h = _mlp(X, w11_ref[...], g1_ref[...], bt1_ref[...],
             w21_ref[...], b21_ref[...], ones)
    X = _agg_concat(h, 2 * IN_C)

    h = _mlp(X, w12_ref[...], g2_ref[...], bt2_ref[...],
             w22_ref[...], b22_ref[...], ones)
    m1 = jnp.max(h.reshape(K, NPB, 4 * IN_C), axis=0)

    pt = jnp.concatenate([m1, m1], axis=1)
    poly_ref[0] = pt

    cid = (lax.broadcasted_iota(jnp.int32, (NPB, 1), 0)
           + pl.program_id(0) * NPB)
    s = jnp.sum(jnp.where(cid < NPOLY, pt * pt, 0.0),
                axis=0, keepdims=True)

    @pl.when(pl.program_id(0) == 0)
    def _():
        csq_ref[...] = s

    @pl.when(pl.program_id(0) != 0)
    def _():
        csq_ref[...] += s


def _attn_kernel(p_ref, csq_ref, m_ref,
                 qw_ref, qb_ref, kw_ref, kb_ref, vw_ref, vb_ref, o_ref):
    p = p_ref[0]
    cn = jnp.sqrt(csq_ref[...])
    pn = p / cn
    rs = jnp.sum(pn * pn, axis=1, keepdims=True)
    pn = pn / jnp.maximum(jnp.sqrt(rs), 1e-12)

    Q = jnp.dot(pn, qw_ref[...], preferred_element_type=F32) + qb_ref[...]
    Kt = jnp.dot(pn, kw_ref[...], preferred_element_type=F32) + kb_ref[...]
    V = jnp.dot(pn, vw_ref[...], preferred_element_type=F32) + vb_ref[...]

    S = lax.dot_general(Q, Kt, (((1,), (1,)), ((), ())),
                        preferred_element_type=F32)
    S = jnp.where(m_ref[0] > 0.5, S, -1e9)
    mx = jnp.max(S, axis=1, keepdims=True)
    E = jnp.exp(S - mx)
    A = E / jnp.sum(E, axis=1, keepdims=True)
    o_ref[0] = jnp.dot(A, V, preferred_element_type=F32)


def kernel(x, edge_index, cluster, valid_len, time_step_len,
           sg0_W1, sg0_b1, sg0_g, sg0_bt, sg0_W2, sg0_b2,
           sg1_W1, sg1_b1, sg1_g, sg1_bt, sg1_W2, sg1_b2,
           sg2_W1, sg2_b1, sg2_g, sg2_bt, sg2_W2, sg2_b2,
           q_W, q_b, k_W, k_b, v_W, v_b):
    xp = jnp.pad(x.reshape(NPOLY, K, IN_C).transpose(1, 0, 2),
                 ((0, 0), (0, NPOLY_P - NPOLY), (0, 1)),
                 constant_values=1.0)

    r2 = lambda a: a.reshape(1, -1)
    wspec = lambda a: pl.BlockSpec(a.shape, lambda b: (0,) * a.ndim)
    ones_m = jnp.full((HID, HID), 1.0 / HID, F32)
    def aw(w, b):
        wb = jnp.concatenate([w, b.reshape(1, -1)], axis=0)
        return jnp.concatenate(
            [wb, jnp.tile(jnp.mean(wb, axis=1, keepdims=True), (1, HID))],
            axis=1)
    weights1 = [aw(sg0_W1, sg0_b1), r2(sg0_g), r2(sg0_bt), sg0_W2, r2(sg0_b2),
                aw(sg1_W1, sg1_b1), r2(sg1_g), r2(sg1_bt), sg1_W2, r2(sg1_b2),
                aw(sg2_W1, sg2_b1), r2(sg2_g), r2(sg2_bt), sg2_W2, r2(sg2_b2)]

    poly, csq = pl.pallas_call(
        _subgraph_kernel,
        grid=(GRID,),
        in_specs=[pl.BlockSpec((K, NPB, IN_C + 1), lambda b: (0, b, 0)),
                  wspec(ones_m)]
                 + [wspec(w) for w in weights1],
        out_specs=[pl.BlockSpec((1, NPB, HID), lambda b: (b, 0, 0)),
                   pl.BlockSpec((1, HID), lambda b: (0, 0))],
        out_shape=[jax.ShapeDtypeStruct((GRID, NPB, HID), F32),
                   jax.ShapeDtypeStruct((1, HID), F32)],
    )(xp, ones_m, *weights1)

    vl = jnp.minimum(valid_len, time_step_len)
    keep = (jnp.arange(T, dtype=jnp.int32)[None, :] < vl[:, None])
    keep = keep.astype(F32).reshape(B, 1, T)

    weights2 = [q_W, r2(q_b), k_W, r2(k_b), v_W, r2(v_b)]
    out = pl.pallas_call(
        _attn_kernel,
        grid=(B,),
        in_specs=[pl.BlockSpec((1, T, HID), lambda b: (b, 0, 0)),
                  pl.BlockSpec((1, HID), lambda b: (0, 0)),
                  pl.BlockSpec((1, 1, T), lambda b: (b, 0, 0))]
                 + [wspec(w) for w in weights2],
        out_specs=pl.BlockSpec((1, T, HID), lambda b: (b, 0, 0)),
        out_shape=jax.ShapeDtypeStruct((B, T, HID), F32),
    )(poly.reshape(NPOLY_P, HID)[:NPOLY].reshape(B, T, HID),
      csq, keep, *weights2)

    return out
